# baseline (speedup 1.0000x reference)
import jax
import jax.numpy as jnp
from jax import lax
from jax.experimental import pallas as pl
from jax.experimental.pallas import tpu as pltpu
from jax.experimental.pallas import tpu_sc as plsc

N_PED = 320000
N_GROUPS = 10000
N_FEAT = 60
NUM_CORES = 2
NUM_SUBCORES = 16
LANES = 16
UNROLL = 5
NPLANES = 4


def _build(n_ped, n_groups, n_feat, interpret=False):
  nw = NUM_CORES * NUM_SUBCORES
  chunk = n_ped // nw
  nv = chunk // LANES
  assert nv % UNROLL == 0
  assert n_feat % 2 == 0
  grow_used = (n_groups + LANES - 1) // LANES
  grow = ((grow_used + 15) // 16) * 16 + 16
  nrow_loc = grow + 16
  acc_rows = (n_feat + 1) * grow
  f_per_s = (n_feat + 1 + NUM_SUBCORES - 1) // NUM_SUBCORES
  gpad = nrow_loc * LANES

  mesh = plsc.VectorSubcoreMesh(
      core_axis_name="c",
      subcore_axis_name="s",
      num_cores=NUM_CORES,
      num_subcores=NUM_SUBCORES,
  )

  def pool_body(
      v_hbm, idx_hbm, out_hbm, idxb, data0, data1, planes, rowbuf, acc_sh,
      sem0, sem1,
  ):
    c = lax.axis_index("c")
    s = lax.axis_index("s")
    wid = s * NUM_CORES + c
    base = pl.multiple_of(wid * chunk, 512)
    iota = lax.iota(jnp.int32, LANES)
    zvec = jnp.zeros((LANES,), jnp.float32)
    pbase = lax.bitwise_and(iota, NPLANES - 1) * gpad

    def zp(j, carry):
      for k in range(8):
        planes[pl.ds((j * 8 + k) * LANES, LANES)] = zvec
      return carry

    lax.fori_loop(0, NPLANES * gpad // (8 * LANES), zp, 0)

    def zl(r, carry):
      rowbuf[r] = zvec
      return carry

    lax.fori_loop(0, nrow_loc, zl, 0)
    for k in range(f_per_s):
      fs = s * f_per_s + k

      @pl.when(fs < n_feat + 1)
      def _():
        pltpu.sync_copy(
            rowbuf.at[pl.ds(0, grow)], acc_sh.at[pl.ds(fs * grow, grow)]
        )

    plsc.subcore_barrier()

    pltpu.sync_copy(idx_hbm.at[pl.ds(base, chunk)], idxb)
    lo = idxb[pl.ds(0, LANES)][0]
    hi = idxb[pl.ds(chunk - LANES, LANES)][LANES - 1]
    row_lo = lax.shift_right_logical(lo, 4)
    row_hi = lax.shift_right_logical(hi, 4)
    ndma = lax.shift_right_logical(row_hi - row_lo, 4) + 1

    def accum_feature(f, datab):
      ones = jnp.ones((LANES,), jnp.float32)

      def body(i, carry):
        for k in range(UNROLL):
          off = (i * UNROLL + k) * LANES
          idxv = idxb[pl.ds(off, LANES)]
          d = datab[pl.ds(off, LANES)] if datab is not None else ones
          plsc.addupdate_scatter(planes, [pbase + idxv], d)
        return carry

      lax.fori_loop(0, nv // UNROLL, body, 0)

      def red(j, carry):
        r0 = row_lo + j * 16
        for k in range(16):
          r = r0 + k
          off = r * LANES
          acc = planes[pl.ds(off, LANES)]
          for p in range(1, NPLANES):
            acc = acc + planes[pl.ds(p * gpad + off, LANES)]
          for p in range(NPLANES):
            planes[pl.ds(p * gpad + off, LANES)] = zvec
          rowbuf[r] = acc
        return carry

      lax.fori_loop(0, ndma, red, 0)

      def dma(j, carry):
        r0 = row_lo + j * 16
        rows = f * grow + r0 + iota
        pltpu.sync_copy(rowbuf.at[pl.ds(r0, 16)], acc_sh.at[rows], add=True)
        return carry

      lax.fori_loop(0, ndma, dma, 0)

    accum_feature(n_feat, None)

    def drain(datab, sem):
      pltpu.make_async_copy(v_hbm.at[0, pl.ds(0, chunk)], datab, sem).wait()

    pltpu.async_copy(v_hbm.at[0, pl.ds(base, chunk)], data0, sem0)

    def fbody(j, carry):
      f0 = j * 2
      pltpu.async_copy(v_hbm.at[f0 + 1, pl.ds(base, chunk)], data1, sem1)
      drain(data0, sem0)
      accum_feature(f0, data0)

      @pl.when(j < n_feat // 2 - 1)
      def _():
        pltpu.async_copy(v_hbm.at[f0 + 2, pl.ds(base, chunk)], data0, sem0)

      drain(data1, sem1)
      accum_feature(f0 + 1, data1)
      return carry

    lax.fori_loop(0, n_feat // 2, fbody, 0)
    plsc.subcore_barrier()

    for k in range(f_per_s):
      fs = s * f_per_s + k

      @pl.when(fs < n_feat + 1)
      def _():
        pltpu.sync_copy(
            acc_sh.at[pl.ds(fs * grow, grow)],
            out_hbm.at[c, pl.ds(fs * grow, grow)],
        )

  pool = pl.kernel(
      pool_body,
      out_type=jax.ShapeDtypeStruct((NUM_CORES, acc_rows, LANES), jnp.float32),
      mesh=mesh,
      scratch_types=[
          pltpu.VMEM((chunk,), jnp.int32),
          pltpu.VMEM((chunk,), jnp.float32),
          pltpu.VMEM((chunk,), jnp.float32),
          pltpu.VMEM((NPLANES * gpad,), jnp.float32),
          pltpu.VMEM((nrow_loc, LANES), jnp.float32),
          pltpu.VMEM_SHARED((acc_rows, LANES), jnp.float32),
          pltpu.SemaphoreType.DMA,
          pltpu.SemaphoreType.DMA,
      ],
      compiler_params=pltpu.CompilerParams(
          use_tc_tiling_on_sc=False, needs_layout_passes=False
      ),
      interpret=interpret,
  )

  def unpool_body(
      part_hbm, idx_hbm, out_hbm, idxb, a0, a1, icnt, pplanes, outb0, outb1,
      semp, semo0, semo1,
  ):
    c = lax.axis_index("c")
    s = lax.axis_index("s")
    wid = s * NUM_CORES + c
    base = pl.multiple_of(wid * chunk, 512)
    iota = lax.iota(jnp.int32, LANES)
    pbase = lax.bitwise_and(iota, NPLANES - 1) * gpad

    pltpu.sync_copy(idx_hbm.at[pl.ds(base, chunk)], idxb)
    lo = idxb[pl.ds(0, LANES)][0]
    hi = idxb[pl.ds(chunk - LANES, LANES)][LANES - 1]
    row_lo = lax.shift_right_logical(lo, 4)
    row_hi = lax.shift_right_logical(hi, 4)
    ndma = lax.shift_right_logical(row_hi - row_lo, 4) + 1

    def ld_rows(f):
      def dj(j, carry):
        r0 = row_lo + j * 16
        pltpu.async_copy(
            part_hbm.at[0, pl.ds(f * grow + r0, 16)], a0.at[pl.ds(r0, 16)],
            semp,
        )
        pltpu.async_copy(
            part_hbm.at[1, pl.ds(f * grow + r0, 16)], a1.at[pl.ds(r0, 16)],
            semp,
        )
        return carry

      lax.fori_loop(0, ndma, dj, 0)

      def dw(j, carry):
        pltpu.make_async_copy(
            part_hbm.at[0, pl.ds(0, 16)], a0.at[pl.ds(0, 16)], semp
        ).wait()
        return carry

      lax.fori_loop(0, 2 * ndma, dw, 0)

    ld_rows(n_feat)

    def ci(j, carry):
      r0 = row_lo + j * 16
      for k in range(16):
        cv = a0[r0 + k] + a1[r0 + k]
        icnt[r0 + k] = 1.0 / jnp.maximum(cv, 1.0)
      return carry

    lax.fori_loop(0, ndma, ci, 0)

    def do_feature(f, outb, semo, first):
      ld_rows(f)

      def pr(j, c2):
        r0 = row_lo + j * 16
        for k in range(16):
          r = r0 + k
          pv = (a0[r] + a1[r]) * icnt[r]
          for p in range(NPLANES):
            pplanes[pl.ds(p * gpad + r * LANES, LANES)] = pv
        return c2

      lax.fori_loop(0, ndma, pr, 0)

      @pl.when(jnp.logical_not(first))
      def _():
        pltpu.make_async_copy(
            outb, out_hbm.at[0, pl.ds(0, chunk)], semo
        ).wait()

      def gb(i, c2):
        for k in range(UNROLL):
          off = (i * UNROLL + k) * LANES
          idxv = idxb[pl.ds(off, LANES)]
          o = plsc.load_gather(pplanes, [pbase + idxv])
          outb[pl.ds(off, LANES)] = o
        return c2

      lax.fori_loop(0, nv // UNROLL, gb, 0)
      pltpu.async_copy(outb, out_hbm.at[f, pl.ds(base, chunk)], semo)

    def fbody(j, carry):
      f0 = j * 2
      do_feature(f0, outb0, semo0, j == 0)
      do_feature(f0 + 1, outb1, semo1, j == 0)
      return carry

    lax.fori_loop(0, n_feat // 2, fbody, 0)
    pltpu.make_async_copy(outb0, out_hbm.at[0, pl.ds(0, chunk)], semo0).wait()
    pltpu.make_async_copy(outb1, out_hbm.at[0, pl.ds(0, chunk)], semo1).wait()

  unpool = pl.kernel(
      unpool_body,
      out_type=jax.ShapeDtypeStruct((n_feat, n_ped), jnp.float32),
      mesh=mesh,
      scratch_types=[
          pltpu.VMEM((chunk,), jnp.int32),
          pltpu.VMEM((nrow_loc, LANES), jnp.float32),
          pltpu.VMEM((nrow_loc, LANES), jnp.float32),
          pltpu.VMEM((nrow_loc, LANES), jnp.float32),
          pltpu.VMEM((NPLANES * gpad,), jnp.float32),
          pltpu.VMEM((chunk,), jnp.float32),
          pltpu.VMEM((chunk,), jnp.float32),
          pltpu.SemaphoreType.DMA,
          pltpu.SemaphoreType.DMA,
          pltpu.SemaphoreType.DMA,
      ],
      compiler_params=pltpu.CompilerParams(
          use_tc_tiling_on_sc=False, needs_layout_passes=False
      ),
      interpret=interpret,
  )

  def run(v, indices):
    n_feat_v = v.shape[1] * v.shape[2]
    v2d = v.reshape(n_feat_v, v.shape[3])
    idx = indices.astype(jnp.int32)
    part = pool(v2d, idx)
    out2d = unpool(part, idx)
    return out2d.reshape(v.shape)

  return run


_run = _build(N_PED, N_GROUPS, N_FEAT)


@jax.jit
def kernel(v, indices):
  return _run(v, indices)


# speedup vs baseline: 1.6845x; 1.6845x over previous
import jax
import jax.numpy as jnp
from jax import lax
from jax.experimental import pallas as pl
from jax.experimental.pallas import tpu as pltpu
from jax.experimental.pallas import tpu_sc as plsc

N_PED = 320000
N_GROUPS = 10000
N_FEAT = 60
NUM_CORES = 2
NUM_SUBCORES = 16
LANES = 16
UNROLL = 5
NPLANES = 4


def _build(n_ped, n_groups, n_feat, interpret=False):
  nw = NUM_CORES * NUM_SUBCORES
  chunk = n_ped // nw
  nv = chunk // LANES
  assert nv % UNROLL == 0
  assert n_feat % 2 == 0
  grow_used = (n_groups + LANES - 1) // LANES
  grow = ((grow_used + 15) // 16) * 16 + 16
  nrow_loc = grow + 16
  acc_rows = (n_feat + 1) * grow
  f_per_s = (n_feat + 1 + NUM_SUBCORES - 1) // NUM_SUBCORES
  gpad = nrow_loc * LANES
  gp2 = gpad + 4

  mesh = plsc.VectorSubcoreMesh(
      core_axis_name="c",
      subcore_axis_name="s",
      num_cores=NUM_CORES,
      num_subcores=NUM_SUBCORES,
  )

  def pool_body(
      v_hbm, idx_hbm, out_hbm, idxb, data0, data1, planes, rowbuf, acc_sh,
      sem0, sem1,
  ):
    c = lax.axis_index("c")
    s = lax.axis_index("s")
    wid = s * NUM_CORES + c
    base = pl.multiple_of(wid * chunk, 512)
    iota = lax.iota(jnp.int32, LANES)
    zvec = jnp.zeros((LANES,), jnp.float32)
    pbase = lax.bitwise_and(iota, NPLANES - 1) * gp2

    def zp(j, carry):
      for k in range(8):
        planes[pl.ds((j * 8 + k) * LANES, LANES)] = zvec
      return carry

    lax.fori_loop(0, (NPLANES * gp2 + 60) // (8 * LANES), zp, 0)

    def zl(r, carry):
      rowbuf[r] = zvec
      return carry

    lax.fori_loop(0, nrow_loc, zl, 0)
    for k in range(f_per_s):
      fs = s * f_per_s + k

      @pl.when(fs < n_feat + 1)
      def _():
        pltpu.sync_copy(
            rowbuf.at[pl.ds(0, grow)], acc_sh.at[pl.ds(fs * grow, grow)]
        )

    plsc.subcore_barrier()

    pltpu.sync_copy(idx_hbm.at[pl.ds(base, chunk)], idxb)
    lo = idxb[pl.ds(0, LANES)][0]
    hi = idxb[pl.ds(chunk - LANES, LANES)][LANES - 1]
    row_lo = lax.shift_right_logical(lo, 4)
    row_hi = lax.shift_right_logical(hi, 4)
    ndma = lax.shift_right_logical(row_hi - row_lo, 4) + 1

    def accum_feature(f, datab):
      ones = jnp.ones((LANES,), jnp.float32)

      def body(i, carry):
        for k in range(UNROLL):
          off = (i * UNROLL + k) * LANES
          idxv = idxb[pl.ds(off, LANES)]
          d = datab[pl.ds(off, LANES)] if datab is not None else ones
          plsc.addupdate_scatter(planes, [pbase + idxv], d)
        return carry

      lax.fori_loop(0, nv // UNROLL, body, 0)

      def red(j, carry):
        r0 = row_lo + j * 16
        for k in range(16):
          r = r0 + k
          off = r * LANES
          acc = planes[pl.ds(off, LANES)]
          for p in range(1, NPLANES):
            acc = acc + planes[pl.ds(p * gp2 + off, LANES)]
          for p in range(NPLANES):
            planes[pl.ds(p * gp2 + off, LANES)] = zvec
          rowbuf[r] = acc
        return carry

      lax.fori_loop(0, ndma, red, 0)

      def dma(j, carry):
        r0 = row_lo + j * 16
        rows = f * grow + r0 + iota
        pltpu.sync_copy(rowbuf.at[pl.ds(r0, 16)], acc_sh.at[rows], add=True)
        return carry

      lax.fori_loop(0, ndma, dma, 0)

    accum_feature(n_feat, None)

    def drain(datab, sem):
      pltpu.make_async_copy(v_hbm.at[0, pl.ds(0, chunk)], datab, sem).wait()

    pltpu.async_copy(v_hbm.at[0, pl.ds(base, chunk)], data0, sem0)

    def fbody(j, carry):
      f0 = j * 2
      pltpu.async_copy(v_hbm.at[f0 + 1, pl.ds(base, chunk)], data1, sem1)
      drain(data0, sem0)
      accum_feature(f0, data0)

      @pl.when(j < n_feat // 2 - 1)
      def _():
        pltpu.async_copy(v_hbm.at[f0 + 2, pl.ds(base, chunk)], data0, sem0)

      drain(data1, sem1)
      accum_feature(f0 + 1, data1)
      return carry

    lax.fori_loop(0, n_feat // 2, fbody, 0)
    plsc.subcore_barrier()

    for k in range(f_per_s):
      fs = s * f_per_s + k

      @pl.when(fs < n_feat + 1)
      def _():
        pltpu.sync_copy(
            acc_sh.at[pl.ds(fs * grow, grow)],
            out_hbm.at[c, pl.ds(fs * grow, grow)],
        )

  pool = pl.kernel(
      pool_body,
      out_type=jax.ShapeDtypeStruct((NUM_CORES, acc_rows, LANES), jnp.float32),
      mesh=mesh,
      scratch_types=[
          pltpu.VMEM((chunk,), jnp.int32),
          pltpu.VMEM((chunk,), jnp.float32),
          pltpu.VMEM((chunk,), jnp.float32),
          pltpu.VMEM((NPLANES * gp2 + 64,), jnp.float32),
          pltpu.VMEM((nrow_loc, LANES), jnp.float32),
          pltpu.VMEM_SHARED((acc_rows, LANES), jnp.float32),
          pltpu.SemaphoreType.DMA,
          pltpu.SemaphoreType.DMA,
      ],
      compiler_params=pltpu.CompilerParams(
          use_tc_tiling_on_sc=False, needs_layout_passes=False
      ),
      interpret=interpret,
  )

  def unpool_body(
      part_hbm, idx_hbm, out_hbm, idxb, a0, a1, icnt, pooled, outb0, outb1,
      semp, semo0, semo1,
  ):
    c = lax.axis_index("c")
    s = lax.axis_index("s")
    wid = s * NUM_CORES + c
    base = pl.multiple_of(wid * chunk, 512)
    pltpu.sync_copy(idx_hbm.at[pl.ds(base, chunk)], idxb)
    lo = idxb[pl.ds(0, LANES)][0]
    hi = idxb[pl.ds(chunk - LANES, LANES)][LANES - 1]
    row_lo = lax.shift_right_logical(lo, 4)
    row_hi = lax.shift_right_logical(hi, 4)
    ndma = lax.shift_right_logical(row_hi - row_lo, 4) + 1

    def ld_rows(f):
      def dj(j, carry):
        r0 = row_lo + j * 16
        pltpu.async_copy(
            part_hbm.at[0, pl.ds(f * grow + r0, 16)], a0.at[pl.ds(r0, 16)],
            semp,
        )
        pltpu.async_copy(
            part_hbm.at[1, pl.ds(f * grow + r0, 16)], a1.at[pl.ds(r0, 16)],
            semp,
        )
        return carry

      lax.fori_loop(0, ndma, dj, 0)

      def dw(j, carry):
        pltpu.make_async_copy(
            part_hbm.at[0, pl.ds(0, 16)], a0.at[pl.ds(0, 16)], semp
        ).wait()
        return carry

      lax.fori_loop(0, 2 * ndma, dw, 0)

    ld_rows(n_feat)

    def ci(j, carry):
      r0 = row_lo + j * 16
      for k in range(16):
        cv = a0[r0 + k] + a1[r0 + k]
        icnt[r0 + k] = 1.0 / jnp.maximum(cv, 1.0)
      return carry

    lax.fori_loop(0, ndma, ci, 0)

    def do_feature(f, outb, semo, first):
      ld_rows(f)

      def pr(j, c2):
        r0 = row_lo + j * 16
        for k in range(16):
          r = r0 + k
          pooled[pl.ds(r * LANES, LANES)] = (a0[r] + a1[r]) * icnt[r]
        return c2

      lax.fori_loop(0, ndma, pr, 0)

      @pl.when(jnp.logical_not(first))
      def _():
        pltpu.make_async_copy(
            outb, out_hbm.at[0, pl.ds(0, chunk)], semo
        ).wait()

      def gb(i, c2):
        for k in range(UNROLL):
          off = (i * UNROLL + k) * LANES
          idxv = idxb[pl.ds(off, LANES)]
          o = plsc.load_gather(pooled, [idxv])
          outb[pl.ds(off, LANES)] = o
        return c2

      lax.fori_loop(0, nv // UNROLL, gb, 0)
      pltpu.async_copy(outb, out_hbm.at[f, pl.ds(base, chunk)], semo)

    def fbody(j, carry):
      f0 = j * 2
      do_feature(f0, outb0, semo0, j == 0)
      do_feature(f0 + 1, outb1, semo1, j == 0)
      return carry

    lax.fori_loop(0, n_feat // 2, fbody, 0)
    pltpu.make_async_copy(outb0, out_hbm.at[0, pl.ds(0, chunk)], semo0).wait()
    pltpu.make_async_copy(outb1, out_hbm.at[0, pl.ds(0, chunk)], semo1).wait()

  unpool = pl.kernel(
      unpool_body,
      out_type=jax.ShapeDtypeStruct((n_feat, n_ped), jnp.float32),
      mesh=mesh,
      scratch_types=[
          pltpu.VMEM((chunk,), jnp.int32),
          pltpu.VMEM((nrow_loc, LANES), jnp.float32),
          pltpu.VMEM((nrow_loc, LANES), jnp.float32),
          pltpu.VMEM((nrow_loc, LANES), jnp.float32),
          pltpu.VMEM((gpad,), jnp.float32),
          pltpu.VMEM((chunk,), jnp.float32),
          pltpu.VMEM((chunk,), jnp.float32),
          pltpu.SemaphoreType.DMA,
          pltpu.SemaphoreType.DMA,
          pltpu.SemaphoreType.DMA,
      ],
      compiler_params=pltpu.CompilerParams(
          use_tc_tiling_on_sc=False, needs_layout_passes=False
      ),
      interpret=interpret,
  )

  def run(v, indices):
    n_feat_v = v.shape[1] * v.shape[2]
    v2d = v.reshape(n_feat_v, v.shape[3])
    idx = indices.astype(jnp.int32)
    part = pool(v2d, idx)
    out2d = unpool(part, idx)
    return out2d.reshape(v.shape)

  return run


_run = _build(N_PED, N_GROUPS, N_FEAT)


@jax.jit
def kernel(v, indices):
  return _run(v, indices)


# speedup vs baseline: 1.8442x; 1.0948x over previous
import jax
import jax.numpy as jnp
from jax import lax
from jax.experimental import pallas as pl
from jax.experimental.pallas import tpu as pltpu
from jax.experimental.pallas import tpu_sc as plsc

N_PED = 320000
N_GROUPS = 10000
N_FEAT = 60
NUM_CORES = 2
NUM_SUBCORES = 16
LANES = 16
UNROLL = 5
NPLANES = 4


def _build(n_ped, n_groups, n_feat, interpret=False):
  nw = NUM_CORES * NUM_SUBCORES
  chunk = n_ped // nw
  nv = chunk // LANES
  assert nv % UNROLL == 0
  assert n_feat % 2 == 0
  grow_used = (n_groups + LANES - 1) // LANES
  grow = ((grow_used + 15) // 16) * 16 + 16
  nrow_loc = grow + 16
  acc_rows = (n_feat + 1) * grow
  f_per_s = (n_feat + 1 + NUM_SUBCORES - 1) // NUM_SUBCORES
  gpad = nrow_loc * LANES
  gp2 = gpad + 4

  mesh = plsc.VectorSubcoreMesh(
      core_axis_name="c",
      subcore_axis_name="s",
      num_cores=NUM_CORES,
      num_subcores=NUM_SUBCORES,
  )

  def pool_body(
      v_hbm, idx_hbm, out_hbm, idxb, data0, data1, planes, rowbuf, acc_sh,
      sem0, sem1, sema,
  ):
    c = lax.axis_index("c")
    s = lax.axis_index("s")
    wid = s * NUM_CORES + c
    base = pl.multiple_of(wid * chunk, 512)
    iota = lax.iota(jnp.int32, LANES)
    zvec = jnp.zeros((LANES,), jnp.float32)
    pbase = lax.bitwise_and(iota, NPLANES - 1) * gp2

    def zp(j, carry):
      for k in range(8):
        planes[pl.ds((j * 8 + k) * LANES, LANES)] = zvec
      return carry

    lax.fori_loop(0, (NPLANES * gp2 + 60) // (8 * LANES), zp, 0)

    def zl(r, carry):
      rowbuf[r] = zvec
      return carry

    lax.fori_loop(0, nrow_loc, zl, 0)
    for k in range(f_per_s):
      fs = s * f_per_s + k

      @pl.when(fs < n_feat + 1)
      def _():
        pltpu.sync_copy(
            rowbuf.at[pl.ds(0, grow)], acc_sh.at[pl.ds(fs * grow, grow)]
        )

    plsc.subcore_barrier()

    pltpu.sync_copy(idx_hbm.at[pl.ds(base, chunk)], idxb)
    lo = idxb[pl.ds(0, LANES)][0]
    hi = idxb[pl.ds(chunk - LANES, LANES)][LANES - 1]
    row_lo = lax.shift_right_logical(lo, 4)
    row_hi = lax.shift_right_logical(hi, 4)
    ndma = lax.shift_right_logical(row_hi - row_lo, 4) + 1

    def pa(i, carry):
      for k in range(UNROLL):
        off = (i * UNROLL + k) * LANES
        idxb[pl.ds(off, LANES)] = idxb[pl.ds(off, LANES)] + pbase
      return carry

    lax.fori_loop(0, nv // UNROLL, pa, 0)

    def accum_feature(f, datab):
      ones = jnp.ones((LANES,), jnp.float32)

      def body(i, carry):
        for k in range(UNROLL):
          off = (i * UNROLL + k) * LANES
          addrv = idxb[pl.ds(off, LANES)]
          d = datab[pl.ds(off, LANES)] if datab is not None else ones
          plsc.addupdate_scatter(planes, [addrv], d)
        return carry

      lax.fori_loop(0, nv // UNROLL, body, 0)

      def red(j, carry):
        r0 = row_lo + j * 16
        for k in range(16):
          r = r0 + k
          off = r * LANES
          acc = planes[pl.ds(off, LANES)]
          for p in range(1, NPLANES):
            acc = acc + planes[pl.ds(p * gp2 + off, LANES)]
          for p in range(NPLANES):
            planes[pl.ds(p * gp2 + off, LANES)] = zvec
          rowbuf[r] = acc
        return carry

      lax.fori_loop(0, ndma, red, 0)

      def dma(j, carry):
        r0 = row_lo + j * 16
        rows = f * grow + r0 + iota
        pltpu.async_copy(
            rowbuf.at[pl.ds(r0, 16)], acc_sh.at[rows], sema, add=True
        )
        return carry

      lax.fori_loop(0, ndma, dma, 0)

      def dwait(j, carry):
        pltpu.make_async_copy(
            rowbuf.at[pl.ds(0, 16)], acc_sh.at[iota], sema
        ).wait()
        return carry

      lax.fori_loop(0, ndma, dwait, 0)

    accum_feature(n_feat, None)

    def drain(datab, sem):
      pltpu.make_async_copy(v_hbm.at[0, pl.ds(0, chunk)], datab, sem).wait()

    pltpu.async_copy(v_hbm.at[0, pl.ds(base, chunk)], data0, sem0)

    def fbody(j, carry):
      f0 = j * 2
      pltpu.async_copy(v_hbm.at[f0 + 1, pl.ds(base, chunk)], data1, sem1)
      drain(data0, sem0)
      accum_feature(f0, data0)

      @pl.when(j < n_feat // 2 - 1)
      def _():
        pltpu.async_copy(v_hbm.at[f0 + 2, pl.ds(base, chunk)], data0, sem0)

      drain(data1, sem1)
      accum_feature(f0 + 1, data1)
      return carry

    lax.fori_loop(0, n_feat // 2, fbody, 0)
    plsc.subcore_barrier()

    for k in range(f_per_s):
      fs = s * f_per_s + k

      @pl.when(fs < n_feat + 1)
      def _():
        pltpu.sync_copy(
            acc_sh.at[pl.ds(fs * grow, grow)],
            out_hbm.at[c, pl.ds(fs * grow, grow)],
        )

  pool = pl.kernel(
      pool_body,
      out_type=jax.ShapeDtypeStruct((NUM_CORES, acc_rows, LANES), jnp.float32),
      mesh=mesh,
      scratch_types=[
          pltpu.VMEM((chunk,), jnp.int32),
          pltpu.VMEM((chunk,), jnp.float32),
          pltpu.VMEM((chunk,), jnp.float32),
          pltpu.VMEM((NPLANES * gp2 + 64,), jnp.float32),
          pltpu.VMEM((nrow_loc, LANES), jnp.float32),
          pltpu.VMEM_SHARED((acc_rows, LANES), jnp.float32),
          pltpu.SemaphoreType.DMA,
          pltpu.SemaphoreType.DMA,
          pltpu.SemaphoreType.DMA,
      ],
      compiler_params=pltpu.CompilerParams(
          use_tc_tiling_on_sc=False, needs_layout_passes=False
      ),
      interpret=interpret,
  )

  def unpool_body(
      part_hbm, idx_hbm, out_hbm, idxb, a0, a1, icnt, pooled, outb0, outb1,
      semp, semo0, semo1,
  ):
    c = lax.axis_index("c")
    s = lax.axis_index("s")
    wid = s * NUM_CORES + c
    base = pl.multiple_of(wid * chunk, 512)
    pltpu.sync_copy(idx_hbm.at[pl.ds(base, chunk)], idxb)
    lo = idxb[pl.ds(0, LANES)][0]
    hi = idxb[pl.ds(chunk - LANES, LANES)][LANES - 1]
    row_lo = lax.shift_right_logical(lo, 4)
    row_hi = lax.shift_right_logical(hi, 4)
    ndma = lax.shift_right_logical(row_hi - row_lo, 4) + 1

    def ld_rows(f):
      def dj(j, carry):
        r0 = row_lo + j * 16
        pltpu.async_copy(
            part_hbm.at[0, pl.ds(f * grow + r0, 16)], a0.at[pl.ds(r0, 16)],
            semp,
        )
        pltpu.async_copy(
            part_hbm.at[1, pl.ds(f * grow + r0, 16)], a1.at[pl.ds(r0, 16)],
            semp,
        )
        return carry

      lax.fori_loop(0, ndma, dj, 0)

      def dw(j, carry):
        pltpu.make_async_copy(
            part_hbm.at[0, pl.ds(0, 16)], a0.at[pl.ds(0, 16)], semp
        ).wait()
        return carry

      lax.fori_loop(0, 2 * ndma, dw, 0)

    ld_rows(n_feat)

    def ci(j, carry):
      r0 = row_lo + j * 16
      for k in range(16):
        cv = a0[r0 + k] + a1[r0 + k]
        icnt[r0 + k] = 1.0 / jnp.maximum(cv, 1.0)
      return carry

    lax.fori_loop(0, ndma, ci, 0)

    def do_feature(f, outb, semo, first):
      ld_rows(f)

      def pr(j, c2):
        r0 = row_lo + j * 16
        for k in range(16):
          r = r0 + k
          pooled[pl.ds(r * LANES, LANES)] = (a0[r] + a1[r]) * icnt[r]
        return c2

      lax.fori_loop(0, ndma, pr, 0)

      @pl.when(jnp.logical_not(first))
      def _():
        pltpu.make_async_copy(
            outb, out_hbm.at[0, pl.ds(0, chunk)], semo
        ).wait()

      def gb(i, c2):
        for k in range(UNROLL):
          off = (i * UNROLL + k) * LANES
          idxv = idxb[pl.ds(off, LANES)]
          o = plsc.load_gather(pooled, [idxv])
          outb[pl.ds(off, LANES)] = o
        return c2

      lax.fori_loop(0, nv // UNROLL, gb, 0)
      pltpu.async_copy(outb, out_hbm.at[f, pl.ds(base, chunk)], semo)

    def fbody(j, carry):
      f0 = j * 2
      do_feature(f0, outb0, semo0, j == 0)
      do_feature(f0 + 1, outb1, semo1, j == 0)
      return carry

    lax.fori_loop(0, n_feat // 2, fbody, 0)
    pltpu.make_async_copy(outb0, out_hbm.at[0, pl.ds(0, chunk)], semo0).wait()
    pltpu.make_async_copy(outb1, out_hbm.at[0, pl.ds(0, chunk)], semo1).wait()

  unpool = pl.kernel(
      unpool_body,
      out_type=jax.ShapeDtypeStruct((n_feat, n_ped), jnp.float32),
      mesh=mesh,
      scratch_types=[
          pltpu.VMEM((chunk,), jnp.int32),
          pltpu.VMEM((nrow_loc, LANES), jnp.float32),
          pltpu.VMEM((nrow_loc, LANES), jnp.float32),
          pltpu.VMEM((nrow_loc, LANES), jnp.float32),
          pltpu.VMEM((gpad,), jnp.float32),
          pltpu.VMEM((chunk,), jnp.float32),
          pltpu.VMEM((chunk,), jnp.float32),
          pltpu.SemaphoreType.DMA,
          pltpu.SemaphoreType.DMA,
          pltpu.SemaphoreType.DMA,
      ],
      compiler_params=pltpu.CompilerParams(
          use_tc_tiling_on_sc=False, needs_layout_passes=False
      ),
      interpret=interpret,
  )

  def run(v, indices):
    n_feat_v = v.shape[1] * v.shape[2]
    v2d = v.reshape(n_feat_v, v.shape[3])
    idx = indices.astype(jnp.int32)
    part = pool(v2d, idx)
    out2d = unpool(part, idx)
    return out2d.reshape(v.shape)

  return run


_run = _build(N_PED, N_GROUPS, N_FEAT)


@jax.jit
def kernel(v, indices):
  return _run(v, indices)


# speedup vs baseline: 1.8674x; 1.0126x over previous
import jax
import jax.numpy as jnp
from jax import lax
from jax.experimental import pallas as pl
from jax.experimental.pallas import tpu as pltpu
from jax.experimental.pallas import tpu_sc as plsc

N_PED = 320000
N_GROUPS = 10000
N_FEAT = 60
NUM_CORES = 2
NUM_SUBCORES = 16
LANES = 16
UNROLL = 25
NPLANES = 4


def _build(n_ped, n_groups, n_feat, interpret=False):
  nw = NUM_CORES * NUM_SUBCORES
  chunk = n_ped // nw
  nv = chunk // LANES
  assert nv % UNROLL == 0
  assert n_feat % 2 == 0
  grow_used = (n_groups + LANES - 1) // LANES
  grow = ((grow_used + 15) // 16) * 16 + 16
  nrow_loc = grow + 16
  acc_rows = (n_feat + 1) * grow
  f_per_s = (n_feat + 1 + NUM_SUBCORES - 1) // NUM_SUBCORES
  gpad = nrow_loc * LANES
  gp2 = gpad + 4

  mesh = plsc.VectorSubcoreMesh(
      core_axis_name="c",
      subcore_axis_name="s",
      num_cores=NUM_CORES,
      num_subcores=NUM_SUBCORES,
  )

  def pool_body(
      v_hbm, idx_hbm, out_hbm, idxb, data0, data1, planes, rowbuf, acc_sh,
      sem0, sem1, sema,
  ):
    c = lax.axis_index("c")
    s = lax.axis_index("s")
    wid = s * NUM_CORES + c
    base = pl.multiple_of(wid * chunk, 512)
    iota = lax.iota(jnp.int32, LANES)
    zvec = jnp.zeros((LANES,), jnp.float32)
    pbase = lax.bitwise_and(iota, NPLANES - 1) * gp2

    def zp(j, carry):
      for k in range(8):
        planes[pl.ds((j * 8 + k) * LANES, LANES)] = zvec
      return carry

    lax.fori_loop(0, (NPLANES * gp2 + 60) // (8 * LANES), zp, 0)

    def zl(r, carry):
      rowbuf[r] = zvec
      return carry

    lax.fori_loop(0, nrow_loc, zl, 0)
    for k in range(f_per_s):
      fs = s * f_per_s + k

      @pl.when(fs < n_feat + 1)
      def _():
        pltpu.sync_copy(
            rowbuf.at[pl.ds(0, grow)], acc_sh.at[pl.ds(fs * grow, grow)]
        )

    plsc.subcore_barrier()

    pltpu.sync_copy(idx_hbm.at[pl.ds(base, chunk)], idxb)
    lo = idxb[pl.ds(0, LANES)][0]
    hi = idxb[pl.ds(chunk - LANES, LANES)][LANES - 1]
    row_lo = lax.shift_right_logical(lo, 4)
    row_hi = lax.shift_right_logical(hi, 4)
    ndma = lax.shift_right_logical(row_hi - row_lo, 4) + 1

    def pa(i, carry):
      for k in range(UNROLL):
        off = (i * UNROLL + k) * LANES
        idxb[pl.ds(off, LANES)] = idxb[pl.ds(off, LANES)] + pbase
      return carry

    lax.fori_loop(0, nv // UNROLL, pa, 0)

    def accum_feature(f, datab):
      ones = jnp.ones((LANES,), jnp.float32)

      def body(i, carry):
        for k in range(UNROLL):
          off = (i * UNROLL + k) * LANES
          addrv = idxb[pl.ds(off, LANES)]
          d = datab[pl.ds(off, LANES)] if datab is not None else ones
          plsc.addupdate_scatter(planes, [addrv], d)
        return carry

      lax.fori_loop(0, nv // UNROLL, body, 0)

      def red(j, carry):
        r0 = row_lo + j * 16
        for k in range(16):
          r = r0 + k
          off = r * LANES
          acc = planes[pl.ds(off, LANES)]
          for p in range(1, NPLANES):
            acc = acc + planes[pl.ds(p * gp2 + off, LANES)]
          for p in range(NPLANES):
            planes[pl.ds(p * gp2 + off, LANES)] = zvec
          rowbuf[r] = acc
        return carry

      lax.fori_loop(0, ndma, red, 0)

      def dma(j, carry):
        r0 = row_lo + j * 16
        rows = f * grow + r0 + iota
        pltpu.async_copy(
            rowbuf.at[pl.ds(r0, 16)], acc_sh.at[rows], sema, add=True
        )
        return carry

      lax.fori_loop(0, ndma, dma, 0)

      def dwait(j, carry):
        pltpu.make_async_copy(
            rowbuf.at[pl.ds(0, 16)], acc_sh.at[iota], sema
        ).wait()
        return carry

      lax.fori_loop(0, ndma, dwait, 0)

    accum_feature(n_feat, None)

    def drain(datab, sem):
      pltpu.make_async_copy(v_hbm.at[0, pl.ds(0, chunk)], datab, sem).wait()

    pltpu.async_copy(v_hbm.at[0, pl.ds(base, chunk)], data0, sem0)

    def fbody(j, carry):
      f0 = j * 2
      pltpu.async_copy(v_hbm.at[f0 + 1, pl.ds(base, chunk)], data1, sem1)
      drain(data0, sem0)
      accum_feature(f0, data0)

      @pl.when(j < n_feat // 2 - 1)
      def _():
        pltpu.async_copy(v_hbm.at[f0 + 2, pl.ds(base, chunk)], data0, sem0)

      drain(data1, sem1)
      accum_feature(f0 + 1, data1)
      return carry

    lax.fori_loop(0, n_feat // 2, fbody, 0)
    plsc.subcore_barrier()

    for k in range(f_per_s):
      fs = s * f_per_s + k

      @pl.when(fs < n_feat + 1)
      def _():
        pltpu.sync_copy(
            acc_sh.at[pl.ds(fs * grow, grow)],
            out_hbm.at[c, pl.ds(fs * grow, grow)],
        )

  pool = pl.kernel(
      pool_body,
      out_type=jax.ShapeDtypeStruct((NUM_CORES, acc_rows, LANES), jnp.float32),
      mesh=mesh,
      scratch_types=[
          pltpu.VMEM((chunk,), jnp.int32),
          pltpu.VMEM((chunk,), jnp.float32),
          pltpu.VMEM((chunk,), jnp.float32),
          pltpu.VMEM((NPLANES * gp2 + 64,), jnp.float32),
          pltpu.VMEM((nrow_loc, LANES), jnp.float32),
          pltpu.VMEM_SHARED((acc_rows, LANES), jnp.float32),
          pltpu.SemaphoreType.DMA,
          pltpu.SemaphoreType.DMA,
          pltpu.SemaphoreType.DMA,
      ],
      compiler_params=pltpu.CompilerParams(
          use_tc_tiling_on_sc=False, needs_layout_passes=False
      ),
      interpret=interpret,
  )

  def unpool_body(
      part_hbm, idx_hbm, out_hbm, idxb, a0, a1, icnt, pooled, outb0, outb1,
      semp, semo0, semo1,
  ):
    c = lax.axis_index("c")
    s = lax.axis_index("s")
    wid = s * NUM_CORES + c
    base = pl.multiple_of(wid * chunk, 512)
    pltpu.sync_copy(idx_hbm.at[pl.ds(base, chunk)], idxb)
    lo = idxb[pl.ds(0, LANES)][0]
    hi = idxb[pl.ds(chunk - LANES, LANES)][LANES - 1]
    row_lo = lax.shift_right_logical(lo, 4)
    row_hi = lax.shift_right_logical(hi, 4)
    ndma = lax.shift_right_logical(row_hi - row_lo, 4) + 1

    def ld_rows(f):
      def dj(j, carry):
        r0 = row_lo + j * 16
        pltpu.async_copy(
            part_hbm.at[0, pl.ds(f * grow + r0, 16)], a0.at[pl.ds(r0, 16)],
            semp,
        )
        pltpu.async_copy(
            part_hbm.at[1, pl.ds(f * grow + r0, 16)], a1.at[pl.ds(r0, 16)],
            semp,
        )
        return carry

      lax.fori_loop(0, ndma, dj, 0)

      def dw(j, carry):
        pltpu.make_async_copy(
            part_hbm.at[0, pl.ds(0, 16)], a0.at[pl.ds(0, 16)], semp
        ).wait()
        return carry

      lax.fori_loop(0, 2 * ndma, dw, 0)

    ld_rows(n_feat)

    def ci(j, carry):
      r0 = row_lo + j * 16
      for k in range(16):
        cv = a0[r0 + k] + a1[r0 + k]
        icnt[r0 + k] = 1.0 / jnp.maximum(cv, 1.0)
      return carry

    lax.fori_loop(0, ndma, ci, 0)

    def do_feature(f, outb, semo, first):
      ld_rows(f)

      def pr(j, c2):
        r0 = row_lo + j * 16
        for k in range(16):
          r = r0 + k
          pooled[pl.ds(r * LANES, LANES)] = (a0[r] + a1[r]) * icnt[r]
        return c2

      lax.fori_loop(0, ndma, pr, 0)

      @pl.when(jnp.logical_not(first))
      def _():
        pltpu.make_async_copy(
            outb, out_hbm.at[0, pl.ds(0, chunk)], semo
        ).wait()

      def gb(i, c2):
        for k in range(UNROLL):
          off = (i * UNROLL + k) * LANES
          idxv = idxb[pl.ds(off, LANES)]
          o = plsc.load_gather(pooled, [idxv])
          outb[pl.ds(off, LANES)] = o
        return c2

      lax.fori_loop(0, nv // UNROLL, gb, 0)
      pltpu.async_copy(outb, out_hbm.at[f, pl.ds(base, chunk)], semo)

    def fbody(j, carry):
      f0 = j * 2
      do_feature(f0, outb0, semo0, j == 0)
      do_feature(f0 + 1, outb1, semo1, j == 0)
      return carry

    lax.fori_loop(0, n_feat // 2, fbody, 0)
    pltpu.make_async_copy(outb0, out_hbm.at[0, pl.ds(0, chunk)], semo0).wait()
    pltpu.make_async_copy(outb1, out_hbm.at[0, pl.ds(0, chunk)], semo1).wait()

  unpool = pl.kernel(
      unpool_body,
      out_type=jax.ShapeDtypeStruct((n_feat, n_ped), jnp.float32),
      mesh=mesh,
      scratch_types=[
          pltpu.VMEM((chunk,), jnp.int32),
          pltpu.VMEM((nrow_loc, LANES), jnp.float32),
          pltpu.VMEM((nrow_loc, LANES), jnp.float32),
          pltpu.VMEM((nrow_loc, LANES), jnp.float32),
          pltpu.VMEM((gpad,), jnp.float32),
          pltpu.VMEM((chunk,), jnp.float32),
          pltpu.VMEM((chunk,), jnp.float32),
          pltpu.SemaphoreType.DMA,
          pltpu.SemaphoreType.DMA,
          pltpu.SemaphoreType.DMA,
      ],
      compiler_params=pltpu.CompilerParams(
          use_tc_tiling_on_sc=False, needs_layout_passes=False
      ),
      interpret=interpret,
  )

  def run(v, indices):
    n_feat_v = v.shape[1] * v.shape[2]
    v2d = v.reshape(n_feat_v, v.shape[3])
    idx = indices.astype(jnp.int32)
    part = pool(v2d, idx)
    out2d = unpool(part, idx)
    return out2d.reshape(v.shape)

  return run


_run = _build(N_PED, N_GROUPS, N_FEAT)


@jax.jit
def kernel(v, indices):
  return _run(v, indices)


# speedup vs baseline: 1.9968x; 1.0693x over previous
import jax
import jax.numpy as jnp
from jax import lax
from jax.experimental import pallas as pl
from jax.experimental.pallas import tpu as pltpu
from jax.experimental.pallas import tpu_sc as plsc

N_PED = 320000
N_GROUPS = 10000
N_FEAT = 60
NUM_CORES = 2
NUM_SUBCORES = 16
LANES = 16
UNROLL = 25
NPLANES = 4


def _build(n_ped, n_groups, n_feat, interpret=False):
  nw = NUM_CORES * NUM_SUBCORES
  chunk = n_ped // nw
  nv = chunk // LANES
  assert nv % UNROLL == 0
  assert n_feat % 2 == 0
  grow_used = (n_groups + LANES - 1) // LANES
  grow = ((grow_used + 15) // 16) * 16 + 16
  nrow_loc = grow + 16
  acc_rows = (n_feat + 1) * grow
  f_per_s = (n_feat + 1 + NUM_SUBCORES - 1) // NUM_SUBCORES
  gpad = nrow_loc * LANES
  gp2 = gpad + 4

  mesh = plsc.VectorSubcoreMesh(
      core_axis_name="c",
      subcore_axis_name="s",
      num_cores=NUM_CORES,
      num_subcores=NUM_SUBCORES,
  )

  def pool_body(
      v_hbm, idx_hbm, out_hbm, idxb, data0, data1, planes, rowbuf, acc_sh,
      sem0, sem1, sema,
  ):
    c = lax.axis_index("c")
    s = lax.axis_index("s")
    wid = s * NUM_CORES + c
    base = pl.multiple_of(wid * chunk, 512)
    iota = lax.iota(jnp.int32, LANES)
    zvec = jnp.zeros((LANES,), jnp.float32)
    pbase = lax.bitwise_and(iota, NPLANES - 1) * gp2

    def zp(j, carry):
      for k in range(8):
        planes[pl.ds((j * 8 + k) * LANES, LANES)] = zvec
      return carry

    lax.fori_loop(0, (NPLANES * gp2 + 60) // (8 * LANES), zp, 0)

    def zl(r, carry):
      rowbuf[r] = zvec
      return carry

    lax.fori_loop(0, nrow_loc, zl, 0)
    for k in range(f_per_s):
      fs = s * f_per_s + k

      @pl.when(fs < n_feat + 1)
      def _():
        pltpu.sync_copy(
            rowbuf.at[pl.ds(0, grow)], acc_sh.at[pl.ds(fs * grow, grow)]
        )

    plsc.subcore_barrier()

    pltpu.sync_copy(idx_hbm.at[pl.ds(base, chunk)], idxb)
    lo = idxb[pl.ds(0, LANES)][0]
    hi = idxb[pl.ds(chunk - LANES, LANES)][LANES - 1]
    row_lo = lax.shift_right_logical(lo, 4)
    row_hi = lax.shift_right_logical(hi, 4)
    ndma = lax.shift_right_logical(row_hi - row_lo, 4) + 1

    def pa(i, carry):
      for k in range(UNROLL):
        off = (i * UNROLL + k) * LANES
        idxb[pl.ds(off, LANES)] = idxb[pl.ds(off, LANES)] + pbase
      return carry

    lax.fori_loop(0, nv // UNROLL, pa, 0)

    def accum_feature(f, datab):
      ones = jnp.ones((LANES,), jnp.float32)

      def body(i, carry):
        for k in range(UNROLL):
          off = (i * UNROLL + k) * LANES
          addrv = idxb[pl.ds(off, LANES)]
          d = datab[pl.ds(off, LANES)] if datab is not None else ones
          plsc.addupdate_scatter(planes, [addrv], d)
        return carry

      lax.fori_loop(0, nv // UNROLL, body, 0)

      def red(j, carry):
        r0 = row_lo + j * 16
        for k in range(16):
          r = r0 + k
          off = r * LANES
          acc = planes[pl.ds(off, LANES)]
          for p in range(1, NPLANES):
            acc = acc + planes[pl.ds(p * gp2 + off, LANES)]
          for p in range(NPLANES):
            planes[pl.ds(p * gp2 + off, LANES)] = zvec
          rowbuf[r] = acc
        return carry

      lax.fori_loop(0, ndma, red, 0)

      def dma(j, carry):
        r0 = row_lo + j * 16
        rows = f * grow + r0 + iota
        pltpu.async_copy(
            rowbuf.at[pl.ds(r0, 16)], acc_sh.at[rows], sema, add=True
        )
        return carry

      lax.fori_loop(0, ndma, dma, 0)

      def dwait(j, carry):
        pltpu.make_async_copy(
            rowbuf.at[pl.ds(0, 16)], acc_sh.at[iota], sema
        ).wait()
        return carry

      lax.fori_loop(0, ndma, dwait, 0)

    accum_feature(n_feat, None)

    def drain(datab, sem):
      pltpu.make_async_copy(v_hbm.at[0, pl.ds(0, chunk)], datab, sem).wait()

    pltpu.async_copy(v_hbm.at[0, pl.ds(base, chunk)], data0, sem0)

    def fbody(j, carry):
      f0 = j * 2
      pltpu.async_copy(v_hbm.at[f0 + 1, pl.ds(base, chunk)], data1, sem1)
      drain(data0, sem0)
      accum_feature(f0, data0)

      @pl.when(j < n_feat // 2 - 1)
      def _():
        pltpu.async_copy(v_hbm.at[f0 + 2, pl.ds(base, chunk)], data0, sem0)

      drain(data1, sem1)
      accum_feature(f0 + 1, data1)
      return carry

    lax.fori_loop(0, n_feat // 2, fbody, 0)
    plsc.subcore_barrier()

    for k in range(f_per_s):
      fs = s * f_per_s + k

      @pl.when(fs < n_feat + 1)
      def _():
        pltpu.sync_copy(
            acc_sh.at[pl.ds(fs * grow, grow)],
            out_hbm.at[c, pl.ds(fs * grow, grow)],
        )

  pool = pl.kernel(
      pool_body,
      out_type=jax.ShapeDtypeStruct((NUM_CORES, acc_rows, LANES), jnp.float32),
      mesh=mesh,
      scratch_types=[
          pltpu.VMEM((chunk,), jnp.int32),
          pltpu.VMEM((chunk,), jnp.float32),
          pltpu.VMEM((chunk,), jnp.float32),
          pltpu.VMEM((NPLANES * gp2 + 64,), jnp.float32),
          pltpu.VMEM((nrow_loc, LANES), jnp.float32),
          pltpu.VMEM_SHARED((acc_rows, LANES), jnp.float32),
          pltpu.SemaphoreType.DMA,
          pltpu.SemaphoreType.DMA,
          pltpu.SemaphoreType.DMA,
      ],
      compiler_params=pltpu.CompilerParams(
          use_tc_tiling_on_sc=False, needs_layout_passes=False
      ),
      interpret=interpret,
  )

  def unpool_body(
      part_hbm, idx_hbm, out_hbm, idxb, a0, a1, b0, b1, icnt, pooled,
      outb0, outb1, semp, semq, semo0, semo1,
  ):
    c = lax.axis_index("c")
    s = lax.axis_index("s")
    wid = s * NUM_CORES + c
    base = pl.multiple_of(wid * chunk, 512)
    pltpu.sync_copy(idx_hbm.at[pl.ds(base, chunk)], idxb)
    lo = idxb[pl.ds(0, LANES)][0]
    hi = idxb[pl.ds(chunk - LANES, LANES)][LANES - 1]
    row_lo = lax.shift_right_logical(lo, 4)
    row_hi = lax.shift_right_logical(hi, 4)
    ndma = lax.shift_right_logical(row_hi - row_lo, 4) + 1

    def issue_rows(f, d0, d1, sem):
      def dj(j, carry):
        r0 = row_lo + j * 16
        pltpu.async_copy(
            part_hbm.at[0, pl.ds(f * grow + r0, 16)], d0.at[pl.ds(r0, 16)],
            sem,
        )
        pltpu.async_copy(
            part_hbm.at[1, pl.ds(f * grow + r0, 16)], d1.at[pl.ds(r0, 16)],
            sem,
        )
        return carry

      lax.fori_loop(0, ndma, dj, 0)

    def drain_rows(d0, sem):
      def dw(j, carry):
        pltpu.make_async_copy(
            part_hbm.at[0, pl.ds(0, 16)], d0.at[pl.ds(0, 16)], sem
        ).wait()
        return carry

      lax.fori_loop(0, 2 * ndma, dw, 0)

    issue_rows(n_feat, a0, a1, semp)
    drain_rows(a0, semp)

    def ci(j, carry):
      r0 = row_lo + j * 16
      for k in range(16):
        cv = a0[r0 + k] + a1[r0 + k]
        icnt[r0 + k] = 1.0 / jnp.maximum(cv, 1.0)
      return carry

    lax.fori_loop(0, ndma, ci, 0)

    def do_feature(f, d0, d1, sem, outb, semo, first):
      drain_rows(d0, sem)

      def pr(j, c2):
        r0 = row_lo + j * 16
        for k in range(16):
          r = r0 + k
          pooled[pl.ds(r * LANES, LANES)] = (d0[r] + d1[r]) * icnt[r]
        return c2

      lax.fori_loop(0, ndma, pr, 0)

      @pl.when(jnp.logical_not(first))
      def _():
        pltpu.make_async_copy(
            outb, out_hbm.at[0, pl.ds(0, chunk)], semo
        ).wait()

      def gb(i, c2):
        for k in range(UNROLL):
          off = (i * UNROLL + k) * LANES
          idxv = idxb[pl.ds(off, LANES)]
          o = plsc.load_gather(pooled, [idxv])
          outb[pl.ds(off, LANES)] = o
        return c2

      lax.fori_loop(0, nv // UNROLL, gb, 0)
      pltpu.async_copy(outb, out_hbm.at[f, pl.ds(base, chunk)], semo)

    issue_rows(0, a0, a1, semp)

    def fbody(j, carry):
      f0 = j * 2
      issue_rows(f0 + 1, b0, b1, semq)
      do_feature(f0, a0, a1, semp, outb0, semo0, j == 0)

      @pl.when(j < n_feat // 2 - 1)
      def _():
        issue_rows(f0 + 2, a0, a1, semp)

      do_feature(f0 + 1, b0, b1, semq, outb1, semo1, j == 0)
      return carry

    lax.fori_loop(0, n_feat // 2, fbody, 0)
    pltpu.make_async_copy(outb0, out_hbm.at[0, pl.ds(0, chunk)], semo0).wait()
    pltpu.make_async_copy(outb1, out_hbm.at[0, pl.ds(0, chunk)], semo1).wait()

  unpool = pl.kernel(
      unpool_body,
      out_type=jax.ShapeDtypeStruct((n_feat, n_ped), jnp.float32),
      mesh=mesh,
      scratch_types=[
          pltpu.VMEM((chunk,), jnp.int32),
          pltpu.VMEM((nrow_loc, LANES), jnp.float32),
          pltpu.VMEM((nrow_loc, LANES), jnp.float32),
          pltpu.VMEM((nrow_loc, LANES), jnp.float32),
          pltpu.VMEM((nrow_loc, LANES), jnp.float32),
          pltpu.VMEM((nrow_loc, LANES), jnp.float32),
          pltpu.VMEM((gpad,), jnp.float32),
          pltpu.VMEM((chunk,), jnp.float32),
          pltpu.VMEM((chunk,), jnp.float32),
          pltpu.SemaphoreType.DMA,
          pltpu.SemaphoreType.DMA,
          pltpu.SemaphoreType.DMA,
          pltpu.SemaphoreType.DMA,
      ],
      compiler_params=pltpu.CompilerParams(
          use_tc_tiling_on_sc=False, needs_layout_passes=False
      ),
      interpret=interpret,
  )

  def run(v, indices):
    n_feat_v = v.shape[1] * v.shape[2]
    v2d = v.reshape(n_feat_v, v.shape[3])
    idx = indices.astype(jnp.int32)
    part = pool(v2d, idx)
    out2d = unpool(part, idx)
    return out2d.reshape(v.shape)

  return run


_run = _build(N_PED, N_GROUPS, N_FEAT)


@jax.jit
def kernel(v, indices):
  return _run(v, indices)
